# Initial kernel scaffold; baseline (speedup 1.0000x reference)
import functools
import math

import jax
import jax.numpy as jnp
from jax import lax
from jax.experimental import pallas as pl
from jax.experimental.pallas import tpu as pltpu
from jax.experimental.pallas import tpu_sc as plsc

_N = 10000
_E = 320000
_F = 128
_K = 6

_NC = 2
_NS = 16
_NW = _NC * _NS
_L = 16

_NP = 10240
_EP = 327680
_CHUNK = 128
_CPW = _EP // _NW // _CHUNK
_CPT = _EP // _NS // _CHUNK
_STRIPE = _NP // _NS
_RPW = _NP // _NW

_BLK = 1280
_NBLK = _NP // _BLK


def _cheb_t(i, x):
    if i == 0:
        return 1.0
    t0, t1 = 1.0, x
    for _ in range(2, i + 1):
        t0, t1 = t1, 2.0 * x * t1 - t0
    return t1


_CMAT = [[2.0 / (_K + 1) * _cheb_t(i, math.cos((_K - j + 0.5) * math.pi / (_K + 1)))
          for j in range(_K + 1)] for i in range(_K + 1)]

_mesh = plsc.VectorSubcoreMesh(core_axis_name="c", subcore_axis_name="s",
                               num_cores=_NC, num_subcores=_NS)


def _deg_dis_body(rowdeg_hbm, dis_hbm, deg_acc, idx_v, ones_v, dbuf_v, rep_v):
    c = lax.axis_index("c")
    s = lax.axis_index("s")
    wid = c * _NS + s

    def _z(i, carry):
        ones_v[pl.ds(i * _L, _L)] = jnp.zeros((_L,), jnp.float32)
        return carry
    lax.fori_loop(0, _CHUNK // _L, _z, 0)
    for j in range(_STRIPE // _CHUNK):
        pltpu.sync_copy(ones_v, deg_acc.at[pl.ds(s * _STRIPE + j * _CHUNK, _CHUNK)])
    plsc.subcore_barrier()

    def _o(i, carry):
        ones_v[pl.ds(i * _L, _L)] = jnp.full((_L,), 1.0, jnp.float32)
        return carry
    lax.fori_loop(0, _CHUNK // _L, _o, 0)
    pltpu.sync_copy(rowdeg_hbm.at[pl.ds(s * _CPT, _CPT)], idx_v)

    def _scat(k, carry):
        pltpu.sync_copy(ones_v, deg_acc.at[idx_v.at[k]], add=True)
        return carry
    lax.fori_loop(0, _CPT, _scat, 0)
    plsc.subcore_barrier()

    pltpu.sync_copy(deg_acc.at[pl.ds(wid * _RPW, _RPW)], dbuf_v)
    half = jnp.full((_L,), 0.5, jnp.float32)
    thalf = jnp.full((_L,), 1.5, jnp.float32)
    magic = jnp.full((_L,), 0x5F3759DF, jnp.int32)
    for i in range(_RPW // _L):
        d = dbuf_v[pl.ds(i * _L, _L)]
        bits = plsc.bitcast(d, jnp.int32)
        y = plsc.bitcast(magic - lax.shift_right_logical(bits, 1), jnp.float32)
        hd = half * d
        for _ in range(3):
            y = y * (thalf - hd * y * y)
        y = jnp.where(d > 0.0, y, jnp.zeros((_L,), jnp.float32))
        dbuf_v[pl.ds(i * _L, _L)] = y
    for r in range(_RPW):
        v = dbuf_v[r] * jnp.ones((_L,), jnp.float32)
        for j in range(_F // _L):
            rep_v[r % _CHUNK, pl.ds(j * _L, _L)] = v
        if r % _CHUNK == _CHUNK - 1:
            pltpu.sync_copy(rep_v, dis_hbm.at[pl.ds(wid * _RPW + (r // _CHUNK) * _CHUNK, _CHUNK)])


def _deg_dis(rowdeg2d):
    return pl.kernel(
        _deg_dis_body,
        out_type=jax.ShapeDtypeStruct((_NP, _F), jnp.float32),
        mesh=_mesh,
        scratch_types=[
            pltpu.VMEM_SHARED((_NP,), jnp.float32),
            pltpu.VMEM((_CPT, _CHUNK), jnp.int32),
            pltpu.VMEM((_CHUNK,), jnp.float32),
            pltpu.VMEM((_RPW,), jnp.float32),
            pltpu.VMEM((_CHUNK, _F), jnp.float32),
        ],
    )(rowdeg2d)


def _prop_body(g_hbm, rowg_hbm, cols_hbm, p_hbm, acc, irow_v, icol_v, rows_v, sem):
    c = lax.axis_index("c")
    s = lax.axis_index("s")
    w = c * _NS + s

    def _z(i, carry):
        for j in range(_F // _L):
            rows_v[i, pl.ds(j * _L, _L)] = jnp.zeros((_L,), jnp.float32)
        return carry
    lax.fori_loop(0, _CHUNK, _z, 0)
    for j in range(_STRIPE // _CHUNK):
        pltpu.sync_copy(rows_v, acc.at[pl.ds(s * _STRIPE + j * _CHUNK, _CHUNK)])
    plsc.subcore_barrier()

    pltpu.sync_copy(rowg_hbm.at[pl.ds(w * _CPW, _CPW)], irow_v)
    pltpu.sync_copy(cols_hbm.at[pl.ds(w * _CPW, _CPW)], icol_v)

    def _edge(k, carry):
        pltpu.async_copy(g_hbm.at[irow_v.at[k]], rows_v, sem).wait()
        pltpu.sync_copy(rows_v, acc.at[icol_v.at[k]], add=True)
        return carry
    lax.fori_loop(0, _CPW, _edge, 0)
    plsc.subcore_barrier()

    pltpu.sync_copy(acc.at[pl.ds(s * _STRIPE, _STRIPE)],
                    p_hbm.at[pl.ds(c * _NP + s * _STRIPE, _STRIPE)])


def _propagate(g, rowg2d, cols2d):
    return pl.kernel(
        _prop_body,
        out_type=jax.ShapeDtypeStruct((_NC * _NP, _F), jnp.float32),
        mesh=_mesh,
        scratch_types=[
            pltpu.VMEM_SHARED((_NP, _F), jnp.float32),
            pltpu.VMEM((_CPW, _CHUNK), jnp.int32),
            pltpu.VMEM((_CPW, _CHUNK), jnp.int32),
            pltpu.VMEM((_CHUNK, _F), jnp.float32),
            pltpu.SemaphoreType.DMA,
        ],
    )(g, rowg2d, cols2d)


def _coe(temp_ref, i):
    acc = 0.0
    for j in range(_K + 1):
        acc = acc + _CMAT[i][j] * jnp.maximum(temp_ref[j], 0.0)
    return acc


def _mlp_body(temp_ref, x_ref, w1t_ref, b1_ref, w2t_ref, b2_ref, dis_ref,
              x2_ref, g0_ref, out0_ref):
    h = jnp.dot(x_ref[...], w1t_ref[...], preferred_element_type=jnp.float32)
    h = jnp.maximum(h + b1_ref[...], 0.0)
    x2 = jnp.dot(h, w2t_ref[...], preferred_element_type=jnp.float32) + b2_ref[...]
    x2_ref[...] = x2
    g0_ref[...] = -dis_ref[...] * x2
    out0_ref[...] = (0.5 * _coe(temp_ref, 0)) * x2


def _mlp_prep(xp, w1t, b1, w2t, b2, dis, temp):
    fs = jax.ShapeDtypeStruct((_NP, _F), jnp.float32)
    return pl.pallas_call(
        _mlp_body,
        grid=(_NBLK,),
        in_specs=[
            pl.BlockSpec(memory_space=pltpu.SMEM),
            pl.BlockSpec((_BLK, _F), lambda i: (i, 0)),
            pl.BlockSpec((_F, _F), lambda i: (0, 0)),
            pl.BlockSpec((1, _F), lambda i: (0, 0)),
            pl.BlockSpec((_F, _F), lambda i: (0, 0)),
            pl.BlockSpec((1, _F), lambda i: (0, 0)),
            pl.BlockSpec((_BLK, _F), lambda i: (i, 0)),
        ],
        out_specs=[pl.BlockSpec((_BLK, _F), lambda i: (i, 0))] * 3,
        out_shape=[fs, fs, fs],
    )(temp, xp, w1t, b1, w2t, b2, dis)


def _combine_body(r, m, s, temp_ref, pa_ref, pb_ref, a_ref, out_ref, dis_ref,
                  tx_ref, g_ref, outn_ref):
    cr = _coe(temp_ref, r)
    dp = dis_ref[...] * (pa_ref[...] + pb_ref[...])
    tx = m * dp - s * a_ref[...] if s else m * dp
    tx_ref[...] = tx
    g_ref[...] = -dis_ref[...] * tx
    outn_ref[...] = out_ref[...] + cr * tx


def _combine(r, m, s, p, a, out, dis, temp):
    fs = jax.ShapeDtypeStruct((_NP, _F), jnp.float32)
    return pl.pallas_call(
        functools.partial(_combine_body, r, m, s),
        grid=(_NBLK,),
        in_specs=[
            pl.BlockSpec(memory_space=pltpu.SMEM),
            pl.BlockSpec((_BLK, _F), lambda i: (i, 0)),
            pl.BlockSpec((_BLK, _F), lambda i: (i + _NBLK, 0)),
            pl.BlockSpec((_BLK, _F), lambda i: (i, 0)),
            pl.BlockSpec((_BLK, _F), lambda i: (i, 0)),
            pl.BlockSpec((_BLK, _F), lambda i: (i, 0)),
        ],
        out_specs=[pl.BlockSpec((_BLK, _F), lambda i: (i, 0))] * 3,
        out_shape=[fs, fs, fs],
    )(temp, p, p, a, out, dis)


def kernel(X, edge_index, W1, b1, W2, b2, temp):
    row = edge_index[0]
    col = edge_index[1]
    pad = _EP - _E
    i32 = jnp.int32
    rowg2d = jnp.concatenate([row, jnp.zeros((pad,), i32)]).reshape(-1, _CHUNK)
    cols2d = jnp.concatenate([col, jnp.full((pad,), _N, i32)]).reshape(-1, _CHUNK)
    rowd2d = jnp.concatenate([row, jnp.full((pad,), _N, i32)]).reshape(-1, _CHUNK)

    dis = _deg_dis(rowd2d)

    xp = jnp.concatenate([X, jnp.zeros((_NP - _N, _F), jnp.float32)])
    x2, g, out = _mlp_prep(xp, W1.T, b1.reshape(1, _F), W2.T, b2.reshape(1, _F),
                           dis, temp)

    tx0 = x2
    p = _propagate(g, rowg2d, cols2d)
    tx1, g, out = _combine(1, 1.0, 0.0, p, tx0, out, dis, temp)
    for r in range(2, _K + 1):
        p = _propagate(g, rowg2d, cols2d)
        tx2, g, out = _combine(r, 2.0, 1.0, p, tx0, out, dis, temp)
        tx0, tx1 = tx1, tx2

    return out[:_N]


# speedup vs baseline: 4.9639x; 4.9639x over previous
import functools
import math

import jax
import jax.numpy as jnp
from jax import lax
from jax.experimental import pallas as pl
from jax.experimental.pallas import tpu as pltpu
from jax.experimental.pallas import tpu_sc as plsc

_N = 10000
_E = 320000
_F = 128
_K = 6

_NC = 2
_NS = 16
_NW = _NC * _NS
_L = 16

_NP = 10240
_EP = 327680
_CHUNK = 128
_CPW = _EP // _NW // _CHUNK
_HCH = 40
_CPT = _EP // _NS // _CHUNK
_STRIPE = _NP // _NS
_RPW = _NP // _NW

_BLK = 1280
_NBLK = _NP // _BLK


def _cheb_t(i, x):
    if i == 0:
        return 1.0
    t0, t1 = 1.0, x
    for _ in range(2, i + 1):
        t0, t1 = t1, 2.0 * x * t1 - t0
    return t1


_CMAT = [[2.0 / (_K + 1) * _cheb_t(i, math.cos((_K - j + 0.5) * math.pi / (_K + 1)))
          for j in range(_K + 1)] for i in range(_K + 1)]

_mesh = plsc.VectorSubcoreMesh(core_axis_name="c", subcore_axis_name="s",
                               num_cores=_NC, num_subcores=_NS)


def _deg_dis_body(rowdeg_hbm, dis_hbm, deg_acc, idx_v, ones_v, dbuf_v, rep_v):
    c = lax.axis_index("c")
    s = lax.axis_index("s")
    wid = c * _NS + s

    def _z(i, carry):
        ones_v[pl.ds(i * _L, _L)] = jnp.zeros((_L,), jnp.float32)
        return carry
    lax.fori_loop(0, _CHUNK // _L, _z, 0)
    for j in range(_STRIPE // _CHUNK):
        pltpu.sync_copy(ones_v, deg_acc.at[pl.ds(s * _STRIPE + j * _CHUNK, _CHUNK)])
    plsc.subcore_barrier()

    def _o(i, carry):
        ones_v[pl.ds(i * _L, _L)] = jnp.full((_L,), 1.0, jnp.float32)
        return carry
    lax.fori_loop(0, _CHUNK // _L, _o, 0)
    pltpu.sync_copy(rowdeg_hbm.at[pl.ds(s * _CPT, _CPT)], idx_v)

    def _scat(k, carry):
        pltpu.sync_copy(ones_v, deg_acc.at[idx_v.at[k]], add=True)
        return carry
    lax.fori_loop(0, _CPT, _scat, 0)
    plsc.subcore_barrier()

    pltpu.sync_copy(deg_acc.at[pl.ds(wid * _RPW, _RPW)], dbuf_v)
    half = jnp.full((_L,), 0.5, jnp.float32)
    thalf = jnp.full((_L,), 1.5, jnp.float32)
    magic = jnp.full((_L,), 0x5F3759DF, jnp.int32)
    for i in range(_RPW // _L):
        d = dbuf_v[pl.ds(i * _L, _L)]
        bits = lax.bitcast_convert_type(d, jnp.int32)
        y = lax.bitcast_convert_type(magic - lax.shift_right_logical(bits, 1), jnp.float32)
        hd = half * d
        for _ in range(3):
            y = y * (thalf - hd * y * y)
        y = jnp.where(d > 0.0, y, jnp.zeros((_L,), jnp.float32))
        for l in range(_L):
            r = i * _L + l
            v = y[l] * jnp.ones((_L,), jnp.float32)
            for j in range(_F // _L):
                rep_v[r % _CHUNK, pl.ds(j * _L, _L)] = v
            if r % _CHUNK == _CHUNK - 1:
                pltpu.sync_copy(rep_v, dis_hbm.at[pl.ds(wid * _RPW + (r // _CHUNK) * _CHUNK, _CHUNK)])


def _deg_dis(rowdeg2d):
    return pl.kernel(
        _deg_dis_body,
        out_type=jax.ShapeDtypeStruct((_NP, _F), jnp.float32),
        mesh=_mesh,
        scratch_types=[
            pltpu.VMEM_SHARED((_NP,), jnp.float32),
            pltpu.VMEM((_CPT, _CHUNK), jnp.int32),
            pltpu.VMEM((_CHUNK,), jnp.float32),
            pltpu.VMEM((_RPW,), jnp.float32),
            pltpu.VMEM((_CHUNK, _F), jnp.float32),
        ],
    )(rowdeg2d)


def _prop_body(g_hbm, rowg_hbm, cols_hbm, p_hbm, acc, irow_v, icol_v,
               rows0, rows1, gs0, gs1):
    c = lax.axis_index("c")
    s = lax.axis_index("s")
    w = c * _NS + s

    def _z(i, carry):
        for j in range(_F // _L):
            rows0[i, pl.ds(j * _L, _L)] = jnp.zeros((_L,), jnp.float32)
        return carry
    lax.fori_loop(0, _CHUNK, _z, 0)
    for j in range(_STRIPE // _CHUNK):
        pltpu.sync_copy(rows0, acc.at[pl.ds(s * _STRIPE + j * _CHUNK, _CHUNK)])
    plsc.subcore_barrier()

    def _gstart(e, buf, sem):
        pltpu.async_copy(g_hbm.at[irow_v.at[e]], buf, sem)

    def _gwait(buf, sem):
        pltpu.make_async_copy(g_hbm.at[pl.ds(0, _CHUNK)], buf, sem).wait()

    for half in range(_CPW // _HCH):
        pltpu.sync_copy(rowg_hbm.at[pl.ds(w * _CPW + half * _HCH, _HCH)], irow_v)
        pltpu.sync_copy(cols_hbm.at[pl.ds(w * _CPW + half * _HCH, _HCH)], icol_v)
        _gstart(0, rows0, gs0)

        def _pair(k2, carry):
            e0 = 2 * k2
            _gwait(rows0, gs0)
            _gstart(e0 + 1, rows1, gs1)
            pltpu.sync_copy(rows0, acc.at[icol_v.at[e0]], add=True)
            _gwait(rows1, gs1)

            @pl.when(e0 + 2 < _HCH)
            def _pf():
                _gstart(e0 + 2, rows0, gs0)

            pltpu.sync_copy(rows1, acc.at[icol_v.at[e0 + 1]], add=True)
            return carry

        lax.fori_loop(0, _HCH // 2, _pair, 0)
    plsc.subcore_barrier()

    pltpu.sync_copy(acc.at[pl.ds(s * _STRIPE, _STRIPE)],
                    p_hbm.at[pl.ds(c * _NP + s * _STRIPE, _STRIPE)])


def _propagate(g, rowg2d, cols2d):
    return pl.kernel(
        _prop_body,
        out_type=jax.ShapeDtypeStruct((_NC * _NP, _F), jnp.float32),
        mesh=_mesh,
        scratch_types=[
            pltpu.VMEM_SHARED((_NP, _F), jnp.float32),
            pltpu.VMEM((_HCH, _CHUNK), jnp.int32),
            pltpu.VMEM((_HCH, _CHUNK), jnp.int32),
            pltpu.VMEM((_CHUNK, _F), jnp.float32),
            pltpu.VMEM((_CHUNK, _F), jnp.float32),
            pltpu.SemaphoreType.DMA,
            pltpu.SemaphoreType.DMA,
        ],
    )(g, rowg2d, cols2d)


def _coe(temp_ref, i):
    acc = 0.0
    for j in range(_K + 1):
        acc = acc + _CMAT[i][j] * jnp.maximum(temp_ref[j], 0.0)
    return acc


def _mlp_body(temp_ref, x_ref, w1t_ref, b1_ref, w2t_ref, b2_ref, dis_ref,
              x2_ref, g0_ref, out0_ref):
    h = jnp.dot(x_ref[...], w1t_ref[...], preferred_element_type=jnp.float32)
    h = jnp.maximum(h + b1_ref[...], 0.0)
    x2 = jnp.dot(h, w2t_ref[...], preferred_element_type=jnp.float32) + b2_ref[...]
    x2_ref[...] = x2
    g0_ref[...] = -dis_ref[...] * x2
    out0_ref[...] = (0.5 * _coe(temp_ref, 0)) * x2


def _mlp_prep(xp, w1t, b1, w2t, b2, dis, temp):
    fs = jax.ShapeDtypeStruct((_NP, _F), jnp.float32)
    return pl.pallas_call(
        _mlp_body,
        grid=(_NBLK,),
        in_specs=[
            pl.BlockSpec(memory_space=pltpu.SMEM),
            pl.BlockSpec((_BLK, _F), lambda i: (i, 0)),
            pl.BlockSpec((_F, _F), lambda i: (0, 0)),
            pl.BlockSpec((1, _F), lambda i: (0, 0)),
            pl.BlockSpec((_F, _F), lambda i: (0, 0)),
            pl.BlockSpec((1, _F), lambda i: (0, 0)),
            pl.BlockSpec((_BLK, _F), lambda i: (i, 0)),
        ],
        out_specs=[pl.BlockSpec((_BLK, _F), lambda i: (i, 0))] * 3,
        out_shape=[fs, fs, fs],
    )(temp, xp, w1t, b1, w2t, b2, dis)


def _combine_body(r, m, s, temp_ref, pa_ref, pb_ref, a_ref, out_ref, dis_ref,
                  tx_ref, g_ref, outn_ref):
    cr = _coe(temp_ref, r)
    dp = dis_ref[...] * (pa_ref[...] + pb_ref[...])
    tx = m * dp - s * a_ref[...] if s else m * dp
    tx_ref[...] = tx
    g_ref[...] = -dis_ref[...] * tx
    outn_ref[...] = out_ref[...] + cr * tx


def _combine(r, m, s, p, a, out, dis, temp):
    fs = jax.ShapeDtypeStruct((_NP, _F), jnp.float32)
    return pl.pallas_call(
        functools.partial(_combine_body, r, m, s),
        grid=(_NBLK,),
        in_specs=[
            pl.BlockSpec(memory_space=pltpu.SMEM),
            pl.BlockSpec((_BLK, _F), lambda i: (i, 0)),
            pl.BlockSpec((_BLK, _F), lambda i: (i + _NBLK, 0)),
            pl.BlockSpec((_BLK, _F), lambda i: (i, 0)),
            pl.BlockSpec((_BLK, _F), lambda i: (i, 0)),
            pl.BlockSpec((_BLK, _F), lambda i: (i, 0)),
        ],
        out_specs=[pl.BlockSpec((_BLK, _F), lambda i: (i, 0))] * 3,
        out_shape=[fs, fs, fs],
    )(temp, p, p, a, out, dis)


def kernel(X, edge_index, W1, b1, W2, b2, temp):
    row = edge_index[0]
    col = edge_index[1]
    pad = _EP - _E
    i32 = jnp.int32
    rowg2d = jnp.concatenate([row, jnp.zeros((pad,), i32)]).reshape(-1, _CHUNK)
    cols2d = jnp.concatenate([col, jnp.full((pad,), _N, i32)]).reshape(-1, _CHUNK)
    rowd2d = jnp.concatenate([row, jnp.full((pad,), _N, i32)]).reshape(-1, _CHUNK)

    dis = _deg_dis(rowd2d)

    xp = jnp.concatenate([X, jnp.zeros((_NP - _N, _F), jnp.float32)])
    x2, g, out = _mlp_prep(xp, W1.T, b1.reshape(1, _F), W2.T, b2.reshape(1, _F),
                           dis, temp)

    tx0 = x2
    p = _propagate(g, rowg2d, cols2d)
    tx1, g, out = _combine(1, 1.0, 0.0, p, tx0, out, dis, temp)
    for r in range(2, _K + 1):
        p = _propagate(g, rowg2d, cols2d)
        tx2, g, out = _combine(r, 2.0, 1.0, p, tx0, out, dis, temp)
        tx0, tx1 = tx1, tx2

    return out[:_N]


# speedup vs baseline: 5.3808x; 1.0840x over previous
import functools
import math

import jax
import jax.numpy as jnp
from jax import lax
from jax.experimental import pallas as pl
from jax.experimental.pallas import tpu as pltpu
from jax.experimental.pallas import tpu_sc as plsc

_N = 10000
_E = 320000
_F = 128
_K = 6

_NC = 2
_NS = 16
_NW = _NC * _NS
_L = 16

_NP = 10240
_EP = 327680
_CHUNK = 128
_CPW = _EP // _NW // _CHUNK
_HCH = 40
_CC0 = 120
_CC1 = 40
_CPT = _EP // _NS // _CHUNK
_STRIPE = _NP // _NS
_RPW = _NP // _NW

_BLK = 1280
_NBLK = _NP // _BLK


def _cheb_t(i, x):
    if i == 0:
        return 1.0
    t0, t1 = 1.0, x
    for _ in range(2, i + 1):
        t0, t1 = t1, 2.0 * x * t1 - t0
    return t1


_CMAT = [[2.0 / (_K + 1) * _cheb_t(i, math.cos((_K - j + 0.5) * math.pi / (_K + 1)))
          for j in range(_K + 1)] for i in range(_K + 1)]

_mesh = plsc.VectorSubcoreMesh(core_axis_name="c", subcore_axis_name="s",
                               num_cores=_NC, num_subcores=_NS)


def _deg_dis_body(rowdeg_hbm, dis_hbm, deg_acc, idx_v, ones_v, dbuf_v, rep_v):
    c = lax.axis_index("c")
    s = lax.axis_index("s")
    wid = c * _NS + s

    def _z(i, carry):
        ones_v[pl.ds(i * _L, _L)] = jnp.zeros((_L,), jnp.float32)
        return carry
    lax.fori_loop(0, _CHUNK // _L, _z, 0)
    for j in range(_STRIPE // _CHUNK):
        pltpu.sync_copy(ones_v, deg_acc.at[pl.ds(s * _STRIPE + j * _CHUNK, _CHUNK)])
    plsc.subcore_barrier()

    def _o(i, carry):
        ones_v[pl.ds(i * _L, _L)] = jnp.full((_L,), 1.0, jnp.float32)
        return carry
    lax.fori_loop(0, _CHUNK // _L, _o, 0)
    pltpu.sync_copy(rowdeg_hbm.at[pl.ds(s * _CPT, _CPT)], idx_v)

    def _scat(k, carry):
        pltpu.sync_copy(ones_v, deg_acc.at[idx_v.at[k]], add=True)
        return carry
    lax.fori_loop(0, _CPT, _scat, 0)
    plsc.subcore_barrier()

    pltpu.sync_copy(deg_acc.at[pl.ds(wid * _RPW, _RPW)], dbuf_v)
    half = jnp.full((_L,), 0.5, jnp.float32)
    thalf = jnp.full((_L,), 1.5, jnp.float32)
    magic = jnp.full((_L,), 0x5F3759DF, jnp.int32)
    for i in range(_RPW // _L):
        d = dbuf_v[pl.ds(i * _L, _L)]
        bits = lax.bitcast_convert_type(d, jnp.int32)
        y = lax.bitcast_convert_type(magic - lax.shift_right_logical(bits, 1), jnp.float32)
        hd = half * d
        for _ in range(3):
            y = y * (thalf - hd * y * y)
        y = jnp.where(d > 0.0, y, jnp.zeros((_L,), jnp.float32))
        for l in range(_L):
            r = i * _L + l
            v = y[l] * jnp.ones((_L,), jnp.float32)
            for j in range(_F // _L):
                rep_v[r % _CHUNK, pl.ds(j * _L, _L)] = v
            if r % _CHUNK == _CHUNK - 1:
                pltpu.sync_copy(rep_v, dis_hbm.at[pl.ds(wid * _RPW + (r // _CHUNK) * _CHUNK, _CHUNK)])


def _deg_dis(rowdeg2d):
    return pl.kernel(
        _deg_dis_body,
        out_type=jax.ShapeDtypeStruct((_NP, _F), jnp.float32),
        mesh=_mesh,
        scratch_types=[
            pltpu.VMEM_SHARED((_NP,), jnp.float32),
            pltpu.VMEM((_CPT, _CHUNK), jnp.int32),
            pltpu.VMEM((_CHUNK,), jnp.float32),
            pltpu.VMEM((_RPW,), jnp.float32),
            pltpu.VMEM((_CHUNK, _F), jnp.float32),
        ],
    )(rowdeg2d)


def _prop_body(g_hbm, rowg_hbm, cols_hbm, p_hbm, acc, irow_v, icol_v,
               rows0, rows1, gs0, gs1):
    c = lax.axis_index("c")
    s = lax.axis_index("s")
    w = c * _NS + s

    def _z(i, carry):
        for j in range(_F // _L):
            rows0[i, pl.ds(j * _L, _L)] = jnp.zeros((_L,), jnp.float32)
        return carry
    lax.fori_loop(0, _CHUNK, _z, 0)
    for j in range(_STRIPE // _CHUNK):
        pltpu.sync_copy(rows0, acc.at[pl.ds(s * _STRIPE + j * _CHUNK, _CHUNK)])
    plsc.subcore_barrier()

    def _gstart(e, buf, sem):
        pltpu.async_copy(g_hbm.at[irow_v.at[e]], buf, sem)

    def _gwait(buf, sem):
        pltpu.make_async_copy(g_hbm.at[pl.ds(0, _CHUNK)], buf, sem).wait()

    def _run(base_chunk, nstage):
        for st_i in range(nstage):
            pltpu.sync_copy(rowg_hbm.at[pl.ds(base_chunk + st_i * _HCH, _HCH)], irow_v)
            pltpu.sync_copy(cols_hbm.at[pl.ds(base_chunk + st_i * _HCH, _HCH)], icol_v)
            _gstart(0, rows0, gs0)

            def _pair(k2, carry):
                e0 = 2 * k2
                _gwait(rows0, gs0)
                _gstart(e0 + 1, rows1, gs1)
                pltpu.sync_copy(rows0, acc.at[icol_v.at[e0]], add=True)
                _gwait(rows1, gs1)

                @pl.when(e0 + 2 < _HCH)
                def _pf():
                    _gstart(e0 + 2, rows0, gs0)

                pltpu.sync_copy(rows1, acc.at[icol_v.at[e0 + 1]], add=True)
                return carry

            lax.fori_loop(0, _HCH // 2, _pair, 0)

    @pl.when(c == 0)
    def _core0():
        _run(s * _CC0, _CC0 // _HCH)

    @pl.when(c == 1)
    def _core1():
        _run(_NS * _CC0 + s * _CC1, _CC1 // _HCH)

    plsc.subcore_barrier()

    pltpu.sync_copy(acc.at[pl.ds(s * _STRIPE, _STRIPE)],
                    p_hbm.at[pl.ds(c * _NP + s * _STRIPE, _STRIPE)])


def _propagate(g, rowg2d, cols2d):
    return pl.kernel(
        _prop_body,
        out_type=jax.ShapeDtypeStruct((_NC * _NP, _F), jnp.float32),
        mesh=_mesh,
        scratch_types=[
            pltpu.VMEM_SHARED((_NP, _F), jnp.float32),
            pltpu.VMEM((_HCH, _CHUNK), jnp.int32),
            pltpu.VMEM((_HCH, _CHUNK), jnp.int32),
            pltpu.VMEM((_CHUNK, _F), jnp.float32),
            pltpu.VMEM((_CHUNK, _F), jnp.float32),
            pltpu.SemaphoreType.DMA,
            pltpu.SemaphoreType.DMA,
        ],
    )(g, rowg2d, cols2d)


def _coe(temp_ref, i):
    acc = 0.0
    for j in range(_K + 1):
        acc = acc + _CMAT[i][j] * jnp.maximum(temp_ref[j], 0.0)
    return acc


def _mlp_body(temp_ref, x_ref, w1t_ref, b1_ref, w2t_ref, b2_ref, dis_ref,
              x2_ref, g0_ref, out0_ref):
    h = jnp.dot(x_ref[...], w1t_ref[...], preferred_element_type=jnp.float32)
    h = jnp.maximum(h + b1_ref[...], 0.0)
    x2 = jnp.dot(h, w2t_ref[...], preferred_element_type=jnp.float32) + b2_ref[...]
    x2_ref[...] = x2
    g0_ref[...] = -dis_ref[...] * x2
    out0_ref[...] = (0.5 * _coe(temp_ref, 0)) * x2


def _mlp_prep(xp, w1t, b1, w2t, b2, dis, temp):
    fs = jax.ShapeDtypeStruct((_NP, _F), jnp.float32)
    return pl.pallas_call(
        _mlp_body,
        grid=(_NBLK,),
        in_specs=[
            pl.BlockSpec(memory_space=pltpu.SMEM),
            pl.BlockSpec((_BLK, _F), lambda i: (i, 0)),
            pl.BlockSpec((_F, _F), lambda i: (0, 0)),
            pl.BlockSpec((1, _F), lambda i: (0, 0)),
            pl.BlockSpec((_F, _F), lambda i: (0, 0)),
            pl.BlockSpec((1, _F), lambda i: (0, 0)),
            pl.BlockSpec((_BLK, _F), lambda i: (i, 0)),
        ],
        out_specs=[pl.BlockSpec((_BLK, _F), lambda i: (i, 0))] * 3,
        out_shape=[fs, fs, fs],
    )(temp, xp, w1t, b1, w2t, b2, dis)


def _combine_body(r, m, s, temp_ref, pa_ref, pb_ref, a_ref, out_ref, dis_ref,
                  tx_ref, g_ref, outn_ref):
    cr = _coe(temp_ref, r)
    dp = dis_ref[...] * (pa_ref[...] + pb_ref[...])
    tx = m * dp - s * a_ref[...] if s else m * dp
    tx_ref[...] = tx
    g_ref[...] = -dis_ref[...] * tx
    outn_ref[...] = out_ref[...] + cr * tx


def _combine(r, m, s, p, a, out, dis, temp):
    fs = jax.ShapeDtypeStruct((_NP, _F), jnp.float32)
    return pl.pallas_call(
        functools.partial(_combine_body, r, m, s),
        grid=(_NBLK,),
        in_specs=[
            pl.BlockSpec(memory_space=pltpu.SMEM),
            pl.BlockSpec((_BLK, _F), lambda i: (i, 0)),
            pl.BlockSpec((_BLK, _F), lambda i: (i + _NBLK, 0)),
            pl.BlockSpec((_BLK, _F), lambda i: (i, 0)),
            pl.BlockSpec((_BLK, _F), lambda i: (i, 0)),
            pl.BlockSpec((_BLK, _F), lambda i: (i, 0)),
        ],
        out_specs=[pl.BlockSpec((_BLK, _F), lambda i: (i, 0))] * 3,
        out_shape=[fs, fs, fs],
    )(temp, p, p, a, out, dis)


def kernel(X, edge_index, W1, b1, W2, b2, temp):
    row = edge_index[0]
    col = edge_index[1]
    pad = _EP - _E
    i32 = jnp.int32
    rowg2d = jnp.concatenate([row, jnp.zeros((pad,), i32)]).reshape(-1, _CHUNK)
    cols2d = jnp.concatenate([col, jnp.full((pad,), _N, i32)]).reshape(-1, _CHUNK)
    rowd2d = jnp.concatenate([row, jnp.full((pad,), _N, i32)]).reshape(-1, _CHUNK)

    dis = _deg_dis(rowd2d)

    xp = jnp.concatenate([X, jnp.zeros((_NP - _N, _F), jnp.float32)])
    x2, g, out = _mlp_prep(xp, W1.T, b1.reshape(1, _F), W2.T, b2.reshape(1, _F),
                           dis, temp)

    tx0 = x2
    p = _propagate(g, rowg2d, cols2d)
    tx1, g, out = _combine(1, 1.0, 0.0, p, tx0, out, dis, temp)
    for r in range(2, _K + 1):
        p = _propagate(g, rowg2d, cols2d)
        tx2, g, out = _combine(r, 2.0, 1.0, p, tx0, out, dis, temp)
        tx0, tx1 = tx1, tx2

    return out[:_N]
